# baseline (speedup 1.0000x reference)
import jax
import jax.numpy as jnp
from jax import lax
from jax.experimental import pallas as pl
from jax.experimental.pallas import tpu as pltpu

_BT = 32


def _dg(a, w):
    return lax.dot_general(a, w, (((1,), (1,)), ((), ())),
                           preferred_element_type=jnp.float32)


def _fused_body(x_ref, wrAB_ref, uA_ref, uB_ref,
                wl0_ref, wl1_ref, wl2_ref, wl3_ref, cA_ref, cB_ref,
                w1A_ref, w1B_ref, cpre_ref, w2_ref, wf_ref, k_ref,
                y_ref, r_ref):
    x3 = x_ref[...]
    xc = x3.reshape(_BT * 128, 128)
    G = _dg(xc.astype(jnp.bfloat16),
            wrAB_ref[...].astype(jnp.bfloat16))
    MA = jnp.sum(x3 * uA_ref[...][None], axis=1)
    MB = jnp.sum(x3 * uB_ref[...][None], axis=1)
    MSGA = 0.5 * (_dg(MB, wl1_ref[...]) + _dg(MA, wl2_ref[...])) + cA_ref[...]
    MSGB = 0.5 * (_dg(MA, wl0_ref[...]) + _dg(MB, wl3_ref[...])) + cB_ref[...]
    w1A = w1A_ref[...]
    w1B = w1B_ref[...]
    cpre = cpre_ref[...]
    for j in range(_BT):
        Gj = G[j * 128:(j + 1) * 128]
        preA = (jnp.dot(w1A, Gj[:, :128], preferred_element_type=jnp.float32)
                + MSGA[j:j + 1, :] + cpre[:64, :])
        preB = (jnp.dot(w1B, Gj[:, 128:], preferred_element_type=jnp.float32)
                + MSGB[j:j + 1, :] + cpre[64:, :])
        r_ref[:64, j * 128:(j + 1) * 128] = jnp.maximum(preA, 0.0)
        r_ref[64:, j * 128:(j + 1) * 128] = jnp.maximum(preB, 0.0)
    t = jnp.dot(w2_ref[...].astype(jnp.bfloat16),
                r_ref[...].astype(jnp.bfloat16),
                preferred_element_type=jnp.float32)
    wf = wf_ref[...]
    k = k_ref[...]
    for j in range(_BT):
        y_ref[j] = _dg(t[:, j * 128:(j + 1) * 128], wf) + k


def kernel(x, W1, b1, W2, b2, sage_Wl, sage_bl, sage_Wr, Wf, bf, period):
    Bb, d_model, Lp, Pp = x.shape
    F = Lp * Pp
    x2 = x.reshape(Bb, d_model, F)

    wrA = 0.5 * (sage_Wr[1] + sage_Wr[2])
    wrB = 0.5 * (sage_Wr[0] + sage_Wr[3])
    wrAB = jnp.concatenate([wrA, wrB], axis=0)
    uA = jnp.mean(W1[:64], axis=0)[:, None] * jnp.ones((1, F), jnp.float32)
    uB = jnp.mean(W1[64:], axis=0)[:, None] * jnp.ones((1, F), jnp.float32)
    bA = jnp.mean(b1[:64])
    bB = jnp.mean(b1[64:])
    cA = (0.5 * (sage_bl[1] + sage_bl[2]
                 + bB * jnp.sum(sage_Wl[1], axis=1)
                 + bA * jnp.sum(sage_Wl[2], axis=1)))[None, :]
    cB = (0.5 * (sage_bl[0] + sage_bl[3]
                 + bA * jnp.sum(sage_Wl[0], axis=1)
                 + bB * jnp.sum(sage_Wl[3], axis=1)))[None, :]
    cpre = jnp.concatenate([
        b1[:64, None] * jnp.sum(wrA, axis=1)[None, :],
        b1[64:, None] * jnp.sum(wrB, axis=1)[None, :]], axis=0)
    k = jnp.sum(W2, axis=1)[:, None] * bf[None, :] + b2[:, None]

    wspec = lambda shp: pl.BlockSpec(shp, lambda b: (0,) * len(shp))
    y2 = pl.pallas_call(
        _fused_body,
        grid=(Bb // _BT,),
        in_specs=[
            pl.BlockSpec((_BT, d_model, F), lambda b: (b, 0, 0)),
            wspec(wrAB.shape),
            wspec(uA.shape),
            wspec(uB.shape),
            wspec(sage_Wl[0].shape),
            wspec(sage_Wl[1].shape),
            wspec(sage_Wl[2].shape),
            wspec(sage_Wl[3].shape),
            wspec(cA.shape),
            wspec(cB.shape),
            wspec(W1[:64].shape),
            wspec(W1[64:].shape),
            wspec(cpre.shape),
            wspec(W2.shape),
            wspec(Wf.shape),
            wspec(k.shape),
        ],
        out_specs=pl.BlockSpec((_BT, W2.shape[0], F), lambda b: (b, 0, 0)),
        out_shape=jax.ShapeDtypeStruct((Bb, W2.shape[0], F), jnp.float32),
        scratch_shapes=[pltpu.VMEM((d_model, _BT * F), jnp.float32)],
        compiler_params=pltpu.CompilerParams(
            dimension_semantics=("parallel",)),
    )(x2, wrAB, uA, uB,
      sage_Wl[0], sage_Wl[1], sage_Wl[2], sage_Wl[3], cA, cB,
      W1[:64], W1[64:], cpre, W2, Wf, k)
    return y2.reshape(Bb, W2.shape[0], Lp, Pp)


# speedup vs baseline: 1.0058x; 1.0058x over previous
import jax
import jax.numpy as jnp
from jax import lax
from jax.experimental import pallas as pl
from jax.experimental.pallas import tpu as pltpu

_BT = 64


def _dg(a, w):
    return lax.dot_general(a, w, (((1,), (1,)), ((), ())),
                           preferred_element_type=jnp.float32)


def _fused_body(x_ref, wrAB_ref, uA_ref, uB_ref,
                wl0_ref, wl1_ref, wl2_ref, wl3_ref, cA_ref, cB_ref,
                w1A_ref, w1B_ref, cpre_ref, w2_ref, wf_ref, k_ref,
                y_ref, r_ref):
    x3 = x_ref[...]
    xc = x3.reshape(_BT * 128, 128)
    G = _dg(xc, wrAB_ref[...])
    MA = jnp.sum(x3 * uA_ref[...][None], axis=1)
    MB = jnp.sum(x3 * uB_ref[...][None], axis=1)
    MSGA = 0.5 * (_dg(MB, wl1_ref[...]) + _dg(MA, wl2_ref[...])) + cA_ref[...]
    MSGB = 0.5 * (_dg(MA, wl0_ref[...]) + _dg(MB, wl3_ref[...])) + cB_ref[...]
    w1A = w1A_ref[...]
    w1B = w1B_ref[...]
    cpre = cpre_ref[...]
    for j in range(_BT):
        Gj = G[j * 128:(j + 1) * 128]
        preA = (jnp.dot(w1A, Gj[:, :128], preferred_element_type=jnp.float32)
                + MSGA[j:j + 1, :] + cpre[:64, :])
        preB = (jnp.dot(w1B, Gj[:, 128:], preferred_element_type=jnp.float32)
                + MSGB[j:j + 1, :] + cpre[64:, :])
        r_ref[:64, j * 128:(j + 1) * 128] = jnp.maximum(preA, 0.0)
        r_ref[64:, j * 128:(j + 1) * 128] = jnp.maximum(preB, 0.0)
    t = jnp.dot(w2_ref[...], r_ref[...],
                preferred_element_type=jnp.float32)
    wf = wf_ref[...]
    k = k_ref[...]
    for j in range(_BT):
        y_ref[j] = _dg(t[:, j * 128:(j + 1) * 128], wf) + k


def kernel(x, W1, b1, W2, b2, sage_Wl, sage_bl, sage_Wr, Wf, bf, period):
    Bb, d_model, Lp, Pp = x.shape
    F = Lp * Pp
    x2 = x.reshape(Bb, d_model, F)

    wrA = 0.5 * (sage_Wr[1] + sage_Wr[2])
    wrB = 0.5 * (sage_Wr[0] + sage_Wr[3])
    wrAB = jnp.concatenate([wrA, wrB], axis=0)
    uA = jnp.mean(W1[:64], axis=0)[:, None] * jnp.ones((1, F), jnp.float32)
    uB = jnp.mean(W1[64:], axis=0)[:, None] * jnp.ones((1, F), jnp.float32)
    bA = jnp.mean(b1[:64])
    bB = jnp.mean(b1[64:])
    cA = (0.5 * (sage_bl[1] + sage_bl[2]
                 + bB * jnp.sum(sage_Wl[1], axis=1)
                 + bA * jnp.sum(sage_Wl[2], axis=1)))[None, :]
    cB = (0.5 * (sage_bl[0] + sage_bl[3]
                 + bA * jnp.sum(sage_Wl[0], axis=1)
                 + bB * jnp.sum(sage_Wl[3], axis=1)))[None, :]
    cpre = jnp.concatenate([
        b1[:64, None] * jnp.sum(wrA, axis=1)[None, :],
        b1[64:, None] * jnp.sum(wrB, axis=1)[None, :]], axis=0)
    k = jnp.sum(W2, axis=1)[:, None] * bf[None, :] + b2[:, None]

    wspec = lambda shp: pl.BlockSpec(shp, lambda b: (0,) * len(shp))
    y2 = pl.pallas_call(
        _fused_body,
        grid=(Bb // _BT,),
        in_specs=[
            pl.BlockSpec((_BT, d_model, F), lambda b: (b, 0, 0)),
            wspec(wrAB.shape),
            wspec(uA.shape),
            wspec(uB.shape),
            wspec(sage_Wl[0].shape),
            wspec(sage_Wl[1].shape),
            wspec(sage_Wl[2].shape),
            wspec(sage_Wl[3].shape),
            wspec(cA.shape),
            wspec(cB.shape),
            wspec(W1[:64].shape),
            wspec(W1[64:].shape),
            wspec(cpre.shape),
            wspec(W2.shape),
            wspec(Wf.shape),
            wspec(k.shape),
        ],
        out_specs=pl.BlockSpec((_BT, W2.shape[0], F), lambda b: (b, 0, 0)),
        out_shape=jax.ShapeDtypeStruct((Bb, W2.shape[0], F), jnp.float32),
        scratch_shapes=[pltpu.VMEM((d_model, _BT * F), jnp.float32)],
        compiler_params=pltpu.CompilerParams(
            dimension_semantics=("parallel",)),
    )(x2, wrAB, uA, uB,
      sage_Wl[0], sage_Wl[1], sage_Wl[2], sage_Wl[3], cA, cB,
      W1[:64], W1[64:], cpre, W2, Wf, k)
    return y2.reshape(Bb, W2.shape[0], Lp, Pp)


# speedup vs baseline: 1.0223x; 1.0165x over previous
import jax
import jax.numpy as jnp
from jax import lax
from jax.experimental import pallas as pl
from jax.experimental.pallas import tpu as pltpu

_BT = 32


def _dg(a, w):
    return lax.dot_general(a, w, (((1,), (1,)), ((), ())),
                           preferred_element_type=jnp.float32)


def _fused_body(x_ref, wrAB_ref, uA_ref, uB_ref,
                wl0_ref, wl1_ref, wl2_ref, wl3_ref, cA_ref, cB_ref,
                w1A_ref, w1B_ref, cpre_ref, w2_ref, wf_ref, k_ref,
                y_ref, g_ref, r_ref, t_ref):
    xc = x_ref[...].reshape(_BT * 128, 128)
    g_ref[...] = _dg(xc, wrAB_ref[...])
    uA = uA_ref[...]
    uB = uB_ref[...]
    mAs, mBs = [], []
    for j in range(_BT):
        xj = x_ref[j]
        mAs.append(jnp.sum(xj * uA, axis=0, keepdims=True))
        mBs.append(jnp.sum(xj * uB, axis=0, keepdims=True))
    MA = jnp.concatenate(mAs, axis=0)
    MB = jnp.concatenate(mBs, axis=0)
    MSGA = 0.5 * (_dg(MB, wl1_ref[...]) + _dg(MA, wl2_ref[...])) + cA_ref[...]
    MSGB = 0.5 * (_dg(MA, wl0_ref[...]) + _dg(MB, wl3_ref[...])) + cB_ref[...]
    w1A = w1A_ref[...]
    w1B = w1B_ref[...]
    cpre = cpre_ref[...]
    for j in range(_BT):
        Gj = g_ref[j * 128:(j + 1) * 128, :]
        preA = (jnp.dot(w1A, Gj[:, :128], preferred_element_type=jnp.float32)
                + MSGA[j:j + 1, :] + cpre[:64, :])
        preB = (jnp.dot(w1B, Gj[:, 128:], preferred_element_type=jnp.float32)
                + MSGB[j:j + 1, :] + cpre[64:, :])
        r_ref[:64, j * 128:(j + 1) * 128] = jnp.maximum(preA, 0.0)
        r_ref[64:, j * 128:(j + 1) * 128] = jnp.maximum(preB, 0.0)
    t_ref[...] = jnp.dot(w2_ref[...], r_ref[...],
                         preferred_element_type=jnp.float32)
    wf = wf_ref[...]
    k = k_ref[...]
    for j in range(_BT):
        y_ref[j] = _dg(t_ref[:, j * 128:(j + 1) * 128], wf) + k


def kernel(x, W1, b1, W2, b2, sage_Wl, sage_bl, sage_Wr, Wf, bf, period):
    Bb, d_model, Lp, Pp = x.shape
    F = Lp * Pp
    x2 = x.reshape(Bb, d_model, F)

    wrA = 0.5 * (sage_Wr[1] + sage_Wr[2])
    wrB = 0.5 * (sage_Wr[0] + sage_Wr[3])
    wrAB = jnp.concatenate([wrA, wrB], axis=0)
    uA = jnp.mean(W1[:64], axis=0)[:, None] * jnp.ones((1, F), jnp.float32)
    uB = jnp.mean(W1[64:], axis=0)[:, None] * jnp.ones((1, F), jnp.float32)
    bA = jnp.mean(b1[:64])
    bB = jnp.mean(b1[64:])
    cA = (0.5 * (sage_bl[1] + sage_bl[2]
                 + bB * jnp.sum(sage_Wl[1], axis=1)
                 + bA * jnp.sum(sage_Wl[2], axis=1)))[None, :]
    cB = (0.5 * (sage_bl[0] + sage_bl[3]
                 + bA * jnp.sum(sage_Wl[0], axis=1)
                 + bB * jnp.sum(sage_Wl[3], axis=1)))[None, :]
    cpre = jnp.concatenate([
        b1[:64, None] * jnp.sum(wrA, axis=1)[None, :],
        b1[64:, None] * jnp.sum(wrB, axis=1)[None, :]], axis=0)
    k = jnp.sum(W2, axis=1)[:, None] * bf[None, :] + b2[:, None]

    wspec = lambda shp: pl.BlockSpec(shp, lambda b: (0,) * len(shp))
    y2 = pl.pallas_call(
        _fused_body,
        grid=(Bb // _BT,),
        in_specs=[
            pl.BlockSpec((_BT, d_model, F), lambda b: (b, 0, 0)),
            wspec(wrAB.shape),
            wspec(uA.shape),
            wspec(uB.shape),
            wspec(sage_Wl[0].shape),
            wspec(sage_Wl[1].shape),
            wspec(sage_Wl[2].shape),
            wspec(sage_Wl[3].shape),
            wspec(cA.shape),
            wspec(cB.shape),
            wspec(W1[:64].shape),
            wspec(W1[64:].shape),
            wspec(cpre.shape),
            wspec(W2.shape),
            wspec(Wf.shape),
            wspec(k.shape),
        ],
        out_specs=pl.BlockSpec((_BT, W2.shape[0], F), lambda b: (b, 0, 0)),
        out_shape=jax.ShapeDtypeStruct((Bb, W2.shape[0], F), jnp.float32),
        scratch_shapes=[pltpu.VMEM((_BT * d_model, 2 * F), jnp.float32),
                        pltpu.VMEM((d_model, _BT * F), jnp.float32),
                        pltpu.VMEM((d_model, _BT * F), jnp.float32)],
        compiler_params=pltpu.CompilerParams(
            dimension_semantics=("parallel",)),
    )(x2, wrAB, uA, uB,
      sage_Wl[0], sage_Wl[1], sage_Wl[2], sage_Wl[3], cA, cB,
      W1[:64], W1[64:], cpre, W2, Wf, k)
    return y2.reshape(Bb, W2.shape[0], Lp, Pp)
